# Initial kernel scaffold; baseline (speedup 1.0000x reference)
import jax
import jax.numpy as jnp
from jax.experimental import pallas as pl


def kernel(Z, positions, neighbors, neighbor_mask, atom_mask, emb, W_in2f, W_fs, W_fp, W_f2out, b_f2out, W_dense, b_dense):
    raise NotImplementedError("write your pallas kernel here")


# speedup vs baseline: 9.5725x; 9.5725x over previous
import functools
import math

import jax
import jax.numpy as jnp
import numpy as np
from jax import lax
from jax.experimental import pallas as pl
from jax.experimental.pallas import tpu as pltpu
from jax.experimental.pallas import tpu_sc as plsc

B, A, NBH, D, K, MAXZ, NL = 2, 2048, 64, 128, 25, 100, 3
CUTOFF = 5.0
LN2 = math.log(2.0)
KPAD = 32
PPAD = 16
N = B * A * NBH
NC, NS = 2, 16
NW = NC * NS
IDX_COLS = 128
IDX_ROWS = N // IDX_COLS
IDX_ROWS_PER_W = IDX_ROWS // NW
BLK = 128
ROWS = BLK * NBH
NBLK = (B * A) // BLK

_kk = np.arange(KPAD)
_logbinom = np.array(
    [math.log(math.comb(K - 1, int(k))) if k < K else -1e30 for k in _kk],
    dtype=np.float32,
).reshape(1, KPAD)


def _sc_gather(table, idx2d, d):
    mesh = plsc.VectorSubcoreMesh(core_axis_name="c", subcore_axis_name="s")

    @functools.partial(
        pl.kernel,
        mesh=mesh,
        out_type=jax.ShapeDtypeStruct((N, d), jnp.float32),
        scratch_types=[
            pltpu.VMEM((IDX_ROWS_PER_W, IDX_COLS), jnp.int32),
            pltpu.VMEM((IDX_COLS, d), jnp.float32),
            pltpu.SemaphoreType.DMA,
        ],
    )
    def k(table_hbm, idx_hbm, out_hbm, idx_v, rows_v, sem):
        wid = lax.axis_index("s") * NC + lax.axis_index("c")
        rbase = wid * IDX_ROWS_PER_W
        pltpu.sync_copy(idx_hbm.at[pl.ds(rbase, IDX_ROWS_PER_W), :], idx_v)

        def body(j, carry):
            pltpu.async_copy(table_hbm.at[idx_v.at[j]], rows_v, sem).wait()
            pltpu.sync_copy(
                rows_v, out_hbm.at[pl.ds((rbase + j) * IDX_COLS, IDX_COLS), :]
            )
            return carry

        lax.fori_loop(0, IDX_ROWS_PER_W, body, 0)

    return k(table, idx2d)


def _prologue_body(z_r, emb_r, w0_r, x_r, xf_r):
    z = z_r[...]
    ids = lax.broadcasted_iota(jnp.int32, (1, 128), 1)
    oh = jnp.where(z == ids, 1.0, 0.0).astype(jnp.float32)
    x = jnp.dot(oh, emb_r[...], preferred_element_type=jnp.float32)
    x_r[...] = x
    xf_r[...] = jnp.dot(x, w0_r[...], preferred_element_type=jnp.float32)


def _prologue(z, emb_pad, w0):
    return pl.pallas_call(
        _prologue_body,
        out_shape=(
            jax.ShapeDtypeStruct((B * A, D), jnp.float32),
            jax.ShapeDtypeStruct((B * A, D), jnp.float32),
        ),
    )(z, emb_pad, w0)


def _geometry_body(pj_r, ps_r, nm_r, lbc_r, gd_r, dirs_r):
    pj = pj_r[:, :PPAD]
    ps = ps_r[...]
    psb = jnp.reshape(
        jnp.broadcast_to(ps[:, None, :], (BLK, NBH, PPAD)), (ROWS, PPAD)
    )
    rvec = pj - psb
    r2 = jnp.sum(rvec * rvec, axis=1, keepdims=True) + 1e-12
    r = jnp.sqrt(r2)
    dirs_r[...] = rvec / r
    fcut = 0.5 * (jnp.cos(jnp.pi * r / CUTOFF) + 1.0)
    fcut = fcut * jnp.where(r < CUTOFF, 1.0, 0.0)
    C = fcut * nm_r[...]
    ex = jnp.exp(-r)
    lx = jnp.log(ex + 1e-10)
    l1x = jnp.log(1.0 - ex + 1e-10)
    kv = lax.broadcasted_iota(jnp.int32, (1, KPAD), 1).astype(jnp.float32)
    fij = jnp.exp(kv * lx + (K - 1.0 - kv) * l1x + lbc_r[...])
    gd_r[...] = fij * C


def _geometry(posj, pos_pad, nmask, lbc):
    return pl.pallas_call(
        _geometry_body,
        grid=(NBLK,),
        in_specs=[
            pl.BlockSpec((ROWS, D), lambda i: (i, 0)),
            pl.BlockSpec((BLK, PPAD), lambda i: (i, 0)),
            pl.BlockSpec((ROWS, 1), lambda i: (i, 0)),
            pl.BlockSpec((1, KPAD), lambda i: (0, 0)),
        ],
        out_specs=(
            pl.BlockSpec((ROWS, KPAD), lambda i: (i, 0)),
            pl.BlockSpec((ROWS, PPAD), lambda i: (i, 0)),
        ),
        out_shape=(
            jax.ShapeDtypeStruct((N, KPAD), jnp.float32),
            jax.ShapeDtypeStruct((N, PPAD), jnp.float32),
        ),
        compiler_params=pltpu.CompilerParams(
            dimension_semantics=("parallel",)
        ),
    )(posj, pos_pad, nmask, lbc)


def _layer_body(last, x_r, yj_r, gd_r, dirs_r, wfs_r, wfp_r, wf2o_r, bf2o_r,
                wd_r, bd_r, wnext_r, xo_r, xf_r=None):
    gd = gd_r[...]
    yj = yj_r[...]
    Ws = jnp.dot(gd, wfs_r[...], preferred_element_type=jnp.float32)
    Wp = jnp.dot(gd, wfp_r[...], preferred_element_type=jnp.float32)
    t = yj * Ws
    s_part = jnp.sum(jnp.reshape(t, (BLK, NBH, D)), axis=1)
    u = yj * Wp
    dirs = dirs_r[...]
    p_part = jnp.zeros((BLK, D), jnp.float32)
    for c in range(3):
        pv = jnp.sum(
            jnp.reshape(u * dirs[:, c:c + 1], (BLK, NBH, D)), axis=1
        )
        p_part = p_part + pv * pv
    y = s_part + p_part
    h = jnp.dot(y, wf2o_r[...], preferred_element_type=jnp.float32) + bf2o_r[...]
    sp = jnp.maximum(h, 0.0) + jnp.log1p(jnp.exp(-jnp.abs(h))) - LN2
    v = jnp.dot(sp, wd_r[...], preferred_element_type=jnp.float32) + bd_r[...]
    xo = x_r[...] + v
    if last:
        xo_r[...] = xo * wnext_r[...]
    else:
        xo_r[...] = xo
        xf_r[...] = jnp.dot(xo, wnext_r[...], preferred_element_type=jnp.float32)


def _layer(x, yj, GD, DIRS, wfs, wfp, wf2o, bf2o, wd, bd, wnext, last):
    wspec = (
        pl.BlockSpec((BLK, 1), lambda i: (i, 0))
        if last
        else pl.BlockSpec((D, D), lambda i: (0, 0))
    )
    out_shape = [jax.ShapeDtypeStruct((B * A, D), jnp.float32)]
    out_specs = [pl.BlockSpec((BLK, D), lambda i: (i, 0))]
    if not last:
        out_shape.append(jax.ShapeDtypeStruct((B * A, D), jnp.float32))
        out_specs.append(pl.BlockSpec((BLK, D), lambda i: (i, 0)))
    res = pl.pallas_call(
        functools.partial(_layer_body, last),
        grid=(NBLK,),
        in_specs=[
            pl.BlockSpec((BLK, D), lambda i: (i, 0)),
            pl.BlockSpec((ROWS, D), lambda i: (i, 0)),
            pl.BlockSpec((ROWS, KPAD), lambda i: (i, 0)),
            pl.BlockSpec((ROWS, PPAD), lambda i: (i, 0)),
            pl.BlockSpec((KPAD, D), lambda i: (0, 0)),
            pl.BlockSpec((KPAD, D), lambda i: (0, 0)),
            pl.BlockSpec((D, D), lambda i: (0, 0)),
            pl.BlockSpec((1, D), lambda i: (0, 0)),
            pl.BlockSpec((D, D), lambda i: (0, 0)),
            pl.BlockSpec((1, D), lambda i: (0, 0)),
            wspec,
        ],
        out_specs=tuple(out_specs),
        out_shape=tuple(out_shape),
        compiler_params=pltpu.CompilerParams(
            dimension_semantics=("parallel",)
        ),
    )(x, yj, GD, DIRS, wfs, wfp, wf2o, bf2o, wd, bd, wnext)
    return res


def kernel(Z, positions, neighbors, neighbor_mask, atom_mask, emb, W_in2f,
           W_fs, W_fp, W_f2out, b_f2out, W_dense, b_dense):
    pos_pad = jnp.pad(
        positions.reshape(B * A, 3).astype(jnp.float32), ((0, 0), (0, PPAD - 3))
    )
    pos_pad128 = jnp.pad(pos_pad, ((0, 0), (0, D - PPAD)))
    offs = (jnp.arange(B, dtype=jnp.int32) * A)[:, None, None]
    idx = (neighbors.astype(jnp.int32) + offs).reshape(IDX_ROWS, IDX_COLS)
    nmask = neighbor_mask.astype(jnp.float32).reshape(N, 1)
    amask = atom_mask.astype(jnp.float32).reshape(B * A, 1)
    lbc = jnp.asarray(_logbinom)
    wfs_p = jnp.pad(W_fs, ((0, 0), (0, KPAD - K), (0, 0)))
    wfp_p = jnp.pad(W_fp, ((0, 0), (0, KPAD - K), (0, 0)))
    emb_pad = jnp.pad(emb.astype(jnp.float32), ((0, 128 - MAXZ), (0, 0)))

    x, xf = _prologue(
        Z.reshape(B * A, 1).astype(jnp.int32), emb_pad, W_in2f[0]
    )
    posj = _sc_gather(pos_pad128, idx, D)
    GD, DIRS = _geometry(posj, pos_pad, nmask, lbc)
    for l in range(NL):
        yj = _sc_gather(xf, idx, D)
        last = l == NL - 1
        wnext = amask if last else W_in2f[l + 1]
        res = _layer(
            x, yj, GD, DIRS, wfs_p[l], wfp_p[l], W_f2out[l],
            b_f2out[l].reshape(1, D), W_dense[l], b_dense[l].reshape(1, D),
            wnext, last,
        )
        if last:
            x = res[0]
        else:
            x, xf = res
    return x.reshape(B, A, D)


# speedup vs baseline: 9.8605x; 1.0301x over previous
import functools
import math

import jax
import jax.numpy as jnp
import numpy as np
from jax import lax
from jax.experimental import pallas as pl
from jax.experimental.pallas import tpu as pltpu
from jax.experimental.pallas import tpu_sc as plsc

B, A, NBH, D, K, MAXZ, NL = 2, 2048, 64, 128, 25, 100, 3
CUTOFF = 5.0
LN2 = math.log(2.0)
KPAD = 32
PPAD = 16
N = B * A * NBH
NC, NS = 2, 16
NW = NC * NS
IDX_COLS = 128
IDX_ROWS = N // IDX_COLS
IDX_ROWS_PER_W = IDX_ROWS // NW
BLK = 128
ROWS = BLK * NBH
NBLK = (B * A) // BLK

_kk = np.arange(KPAD)
_logbinom = np.array(
    [math.log(math.comb(K - 1, int(k))) if k < K else -1e30 for k in _kk],
    dtype=np.float32,
).reshape(1, KPAD)


def _sc_gather(table, idx2d, d, dtype):
    mesh = plsc.VectorSubcoreMesh(core_axis_name="c", subcore_axis_name="s")
    nj = IDX_ROWS_PER_W

    @functools.partial(
        pl.kernel,
        mesh=mesh,
        out_type=jax.ShapeDtypeStruct((N, d), dtype),
        scratch_types=[
            pltpu.VMEM((nj, IDX_COLS), jnp.int32),
            pltpu.VMEM((IDX_COLS, d), dtype),
            pltpu.VMEM((IDX_COLS, d), dtype),
            pltpu.SemaphoreType.DMA,
            pltpu.SemaphoreType.DMA,
        ],
    )
    def k(table_hbm, idx_hbm, out_hbm, idx_v, rows0, rows1, sem0, sem1):
        wid = lax.axis_index("s") * NC + lax.axis_index("c")
        rbase = wid * nj
        pltpu.sync_copy(idx_hbm.at[pl.ds(rbase, nj), :], idx_v)
        pltpu.make_async_copy(table_hbm.at[idx_v.at[0]], rows0, sem0).start()

        def body(i, carry):
            j0 = 2 * i
            pltpu.make_async_copy(
                table_hbm.at[idx_v.at[j0 + 1]], rows1, sem1
            ).start()
            pltpu.make_async_copy(
                out_hbm.at[pl.ds(0, IDX_COLS), :], rows0, sem0
            ).wait()
            pltpu.sync_copy(
                rows0, out_hbm.at[pl.ds((rbase + j0) * IDX_COLS, IDX_COLS), :]
            )
            jn = jnp.minimum(j0 + 2, nj - 1)
            pltpu.make_async_copy(
                table_hbm.at[idx_v.at[jn]], rows0, sem0
            ).start()
            pltpu.make_async_copy(
                out_hbm.at[pl.ds(0, IDX_COLS), :], rows1, sem1
            ).wait()
            pltpu.sync_copy(
                rows1,
                out_hbm.at[pl.ds((rbase + j0 + 1) * IDX_COLS, IDX_COLS), :],
            )
            return carry

        lax.fori_loop(0, nj // 2, body, 0)
        pltpu.make_async_copy(
            out_hbm.at[pl.ds(0, IDX_COLS), :], rows0, sem0
        ).wait()

    return k(table, idx2d)


def _prologue_body(z_r, emb_r, w0_r, x_r, xf_r):
    z = z_r[...]
    ids = lax.broadcasted_iota(jnp.int32, (1, 128), 1)
    oh = jnp.where(z == ids, 1.0, 0.0).astype(jnp.float32)
    x = jnp.dot(oh, emb_r[...], preferred_element_type=jnp.float32)
    x_r[...] = x
    xf_r[...] = jnp.dot(x, w0_r[...], preferred_element_type=jnp.float32)


def _prologue(z, emb_pad, w0):
    return pl.pallas_call(
        _prologue_body,
        out_shape=(
            jax.ShapeDtypeStruct((B * A, D), jnp.float32),
            jax.ShapeDtypeStruct((B * A, D), jnp.float32),
        ),
    )(z, emb_pad, w0)


def _geometry_body(pj_r, ps_r, nm_r, lbc_r, gd_r, dirs_r):
    pj = pj_r[:, :PPAD]
    ps = ps_r[...]
    psb = jnp.reshape(
        jnp.broadcast_to(ps[:, None, :], (BLK, NBH, PPAD)), (ROWS, PPAD)
    )
    rvec = pj - psb
    r2 = jnp.sum(rvec * rvec, axis=1, keepdims=True) + 1e-12
    r = jnp.sqrt(r2)
    dirs_r[...] = rvec / r
    fcut = 0.5 * (jnp.cos(jnp.pi * r / CUTOFF) + 1.0)
    fcut = fcut * jnp.where(r < CUTOFF, 1.0, 0.0)
    C = fcut * nm_r[...]
    ex = jnp.exp(-r)
    lx = jnp.log(ex + 1e-10)
    l1x = jnp.log(1.0 - ex + 1e-10)
    kv = lax.broadcasted_iota(jnp.int32, (1, KPAD), 1).astype(jnp.float32)
    fij = jnp.exp(kv * lx + (K - 1.0 - kv) * l1x + lbc_r[...])
    gd_r[...] = fij * C


def _geometry(posj, pos_pad, nmask, lbc):
    return pl.pallas_call(
        _geometry_body,
        grid=(NBLK,),
        in_specs=[
            pl.BlockSpec((ROWS, D), lambda i: (i, 0)),
            pl.BlockSpec((BLK, PPAD), lambda i: (i, 0)),
            pl.BlockSpec((ROWS, 1), lambda i: (i, 0)),
            pl.BlockSpec((1, KPAD), lambda i: (0, 0)),
        ],
        out_specs=(
            pl.BlockSpec((ROWS, KPAD), lambda i: (i, 0)),
            pl.BlockSpec((ROWS, PPAD), lambda i: (i, 0)),
        ),
        out_shape=(
            jax.ShapeDtypeStruct((N, KPAD), jnp.float32),
            jax.ShapeDtypeStruct((N, PPAD), jnp.float32),
        ),
        compiler_params=pltpu.CompilerParams(
            dimension_semantics=("parallel",)
        ),
    )(posj, pos_pad, nmask, lbc)


def _layer_body(last, x_r, yj_r, gd_r, dirs_r, wfs_r, wfp_r, wf2o_r, bf2o_r,
                wd_r, bd_r, wnext_r, xo_r, xf_r=None):
    gd = gd_r[...]
    yj = yj_r[...]
    Ws = jnp.dot(gd, wfs_r[...], preferred_element_type=jnp.float32)
    Wp = jnp.dot(gd, wfp_r[...], preferred_element_type=jnp.float32)
    t = yj * Ws
    s_part = jnp.sum(jnp.reshape(t, (BLK, NBH, D)), axis=1)
    u = yj * Wp
    dirs = dirs_r[...]
    p_part = jnp.zeros((BLK, D), jnp.float32)
    for c in range(3):
        pv = jnp.sum(
            jnp.reshape(u * dirs[:, c:c + 1], (BLK, NBH, D)), axis=1
        )
        p_part = p_part + pv * pv
    y = s_part + p_part
    h = jnp.dot(y, wf2o_r[...], preferred_element_type=jnp.float32) + bf2o_r[...]
    sp = jnp.maximum(h, 0.0) + jnp.log1p(jnp.exp(-jnp.abs(h))) - LN2
    v = jnp.dot(sp, wd_r[...], preferred_element_type=jnp.float32) + bd_r[...]
    xo = x_r[...] + v
    if last:
        xo_r[...] = xo * wnext_r[...]
    else:
        xo_r[...] = xo
        xf_r[...] = jnp.dot(xo, wnext_r[...], preferred_element_type=jnp.float32)


def _layer(x, yj, GD, DIRS, wfs, wfp, wf2o, bf2o, wd, bd, wnext, last):
    wspec = (
        pl.BlockSpec((BLK, 1), lambda i: (i, 0))
        if last
        else pl.BlockSpec((D, D), lambda i: (0, 0))
    )
    out_shape = [jax.ShapeDtypeStruct((B * A, D), jnp.float32)]
    out_specs = [pl.BlockSpec((BLK, D), lambda i: (i, 0))]
    if not last:
        out_shape.append(jax.ShapeDtypeStruct((B * A, D), jnp.float32))
        out_specs.append(pl.BlockSpec((BLK, D), lambda i: (i, 0)))
    res = pl.pallas_call(
        functools.partial(_layer_body, last),
        grid=(NBLK,),
        in_specs=[
            pl.BlockSpec((BLK, D), lambda i: (i, 0)),
            pl.BlockSpec((ROWS, D), lambda i: (i, 0)),
            pl.BlockSpec((ROWS, KPAD), lambda i: (i, 0)),
            pl.BlockSpec((ROWS, PPAD), lambda i: (i, 0)),
            pl.BlockSpec((KPAD, D), lambda i: (0, 0)),
            pl.BlockSpec((KPAD, D), lambda i: (0, 0)),
            pl.BlockSpec((D, D), lambda i: (0, 0)),
            pl.BlockSpec((1, D), lambda i: (0, 0)),
            pl.BlockSpec((D, D), lambda i: (0, 0)),
            pl.BlockSpec((1, D), lambda i: (0, 0)),
            wspec,
        ],
        out_specs=tuple(out_specs),
        out_shape=tuple(out_shape),
        compiler_params=pltpu.CompilerParams(
            dimension_semantics=("parallel",)
        ),
    )(x, yj, GD, DIRS, wfs, wfp, wf2o, bf2o, wd, bd, wnext)
    return res


def kernel(Z, positions, neighbors, neighbor_mask, atom_mask, emb, W_in2f,
           W_fs, W_fp, W_f2out, b_f2out, W_dense, b_dense):
    pos_pad = jnp.pad(
        positions.reshape(B * A, 3).astype(jnp.float32), ((0, 0), (0, PPAD - 3))
    )
    pos_pad128 = jnp.pad(pos_pad, ((0, 0), (0, D - PPAD)))
    offs = (jnp.arange(B, dtype=jnp.int32) * A)[:, None, None]
    idx = (neighbors.astype(jnp.int32) + offs).reshape(IDX_ROWS, IDX_COLS)
    nmask = neighbor_mask.astype(jnp.float32).reshape(N, 1)
    amask = atom_mask.astype(jnp.float32).reshape(B * A, 1)
    lbc = jnp.asarray(_logbinom)
    wfs_p = jnp.pad(W_fs, ((0, 0), (0, KPAD - K), (0, 0)))
    wfp_p = jnp.pad(W_fp, ((0, 0), (0, KPAD - K), (0, 0)))
    emb_pad = jnp.pad(emb.astype(jnp.float32), ((0, 128 - MAXZ), (0, 0)))

    x, xf = _prologue(
        Z.reshape(B * A, 1).astype(jnp.int32), emb_pad, W_in2f[0]
    )
    posj = _sc_gather(pos_pad128, idx, D, jnp.float32)
    GD, DIRS = _geometry(posj, pos_pad, nmask, lbc)
    for l in range(NL):
        yj = _sc_gather(xf, idx, D, jnp.float32)
        last = l == NL - 1
        wnext = amask if last else W_in2f[l + 1]
        res = _layer(
            x, yj, GD, DIRS, wfs_p[l], wfp_p[l], W_f2out[l],
            b_f2out[l].reshape(1, D), W_dense[l], b_dense[l].reshape(1, D),
            wnext, last,
        )
        if last:
            x = res[0]
        else:
            x, xf = res
    return x.reshape(B, A, D)
